# baseline (speedup 1.0000x reference)
import functools

import jax
import jax.numpy as jnp
from jax import lax
from jax.experimental import pallas as pl
from jax.experimental.pallas import tpu as pltpu

N_NODES = None


def _ln(x, g, b):
    mu = jnp.mean(x, axis=-1, keepdims=True)
    v = jnp.var(x, axis=-1, keepdims=True)
    return (x - mu) / jnp.sqrt(v + 1e-5) * g + b


def _mish(x):
    return x * jnp.tanh(jax.nn.softplus(x))


def _pick_block(n, candidates):
    for c in candidates:
        if n % c == 0:
            return c
    return n


def _node_mlp_body(x_ref, w1_ref, b1_ref, g1_ref, e1_ref,
                   w2_ref, b2_ref, g2_ref, e2_ref, o_ref):
    h = jnp.dot(x_ref[...], w1_ref[...], preferred_element_type=jnp.float32, precision=lax.Precision.HIGHEST)
    h = _ln(h + b1_ref[...], g1_ref[...], e1_ref[...])
    h = jax.nn.relu(h)
    h = jnp.dot(h, w2_ref[...], preferred_element_type=jnp.float32, precision=lax.Precision.HIGHEST)
    h = _ln(h + b2_ref[...], g2_ref[...], e2_ref[...])
    o_ref[...] = _mish(h)


def _node_mlp(node_x, p):
    n, din = node_x.shape
    d1 = p['nae_w1'].shape[1]
    bn = _pick_block(n, (1000, 500, 200, 100, 8))
    grid = (n // bn,)
    full = lambda a: pl.BlockSpec(a.shape, lambda i: (0,) * a.ndim)
    vecs = [p['nae_b1'].reshape(1, -1), p['nae_g1'].reshape(1, -1),
            p['nae_e1'].reshape(1, -1), p['nae_w2'],
            p['nae_b2'].reshape(1, -1), p['nae_g2'].reshape(1, -1),
            p['nae_e2'].reshape(1, -1)]
    return pl.pallas_call(
        _node_mlp_body,
        grid=grid,
        in_specs=[pl.BlockSpec((bn, din), lambda i: (i, 0)), full(p['nae_w1'])]
                 + [full(v) for v in vecs],
        out_specs=pl.BlockSpec((bn, d1), lambda i: (i, 0)),
        out_shape=jax.ShapeDtypeStruct((n, d1), jnp.float32),
    )(node_x, p['nae_w1'], *vecs)


def _conv_body(x_ref, w_ref, sc_ref, sh_ref, o_ref, st_ref, *, cin, rows,
               activate):
    i = pl.program_id(0)
    a = x_ref[...]
    if activate:
        a = _mish(a * sc_ref[...] + sh_ref[...])
    p = lax.broadcasted_iota(jnp.int32, (rows, 1), 0) % 49
    y = p // 7
    x = p % 7
    parts = []
    for dy in range(3):
        for dx in range(3):
            s = 7 * (dy - 1) + (dx - 1)
            yy = y + (dy - 1)
            xx = x + (dx - 1)
            valid = (yy >= 0) & (yy < 7) & (xx >= 0) & (xx < 7)
            mask = valid.astype(jnp.float32)
            shifted = jnp.roll(a, -s, axis=0) if s != 0 else a
            parts.append(shifted * mask)
    cat = jnp.concatenate(parts, axis=1)
    out = jnp.dot(cat, w_ref[...], preferred_element_type=jnp.float32, precision=lax.Precision.HIGHEST)
    o_ref[...] = out
    cout = out.shape[1]
    js = jnp.sum(out, axis=0)
    jss = jnp.sum(out * out, axis=0)
    row = lax.broadcasted_iota(jnp.int32, (8, cout), 0)
    blockstat = (jnp.where(row == 0, jnp.broadcast_to(js[None, :], (8, cout)), 0.0)
                 + jnp.where(row == 1, jnp.broadcast_to(jss[None, :], (8, cout)), 0.0))

    @pl.when(i == 0)
    def _():
        st_ref[...] = jnp.zeros_like(st_ref)

    st_ref[...] += blockstat


def _conv_layer(x_flat, wmat, scale, shift, cout, activate):
    rows_total, cin = x_flat.shape
    n = rows_total // 49
    bn = _pick_block(n, (80, 40, 8))
    rows = bn * 49
    grid = (n // bn,)
    sc = scale.reshape(1, -1)
    sh = shift.reshape(1, -1)
    body = functools.partial(_conv_body, cin=cin, rows=rows, activate=activate)
    out, st = pl.pallas_call(
        body,
        grid=grid,
        in_specs=[pl.BlockSpec((rows, cin), lambda i: (i, 0)),
                  pl.BlockSpec(wmat.shape, lambda i: (0, 0)),
                  pl.BlockSpec(sc.shape, lambda i: (0, 0)),
                  pl.BlockSpec(sh.shape, lambda i: (0, 0))],
        out_specs=[pl.BlockSpec((rows, cout), lambda i: (i, 0)),
                   pl.BlockSpec((8, cout), lambda i: (0, 0))],
        out_shape=[jax.ShapeDtypeStruct((rows_total, cout), jnp.float32),
                   jax.ShapeDtypeStruct((8, cout), jnp.float32)],
    )(x_flat, wmat, sc, sh)
    return out, st


def _bn_affine(st, g, b, count):
    mu = st[0] / count
    var = st[1] / count - mu * mu
    scale = g / jnp.sqrt(var + 1e-5)
    shift = b - mu * scale
    return scale, shift


def _pool_body(x_ref, sc_ref, sh_ref, o_ref, *, bn, rows):
    a = _mish(x_ref[...] * sc_ref[...] + sh_ref[...])
    ni = lax.broadcasted_iota(jnp.int32, (bn, rows), 0)
    ri = lax.broadcasted_iota(jnp.int32, (bn, rows), 1)
    pm = jnp.where(ri // 49 == ni, 1.0 / 49.0, 0.0)
    o_ref[...] = jnp.dot(pm, a, preferred_element_type=jnp.float32, precision=lax.Precision.HIGHEST)


def _pool(x_flat, scale, shift):
    rows_total, c = x_flat.shape
    n = rows_total // 49
    bn = _pick_block(n, (80, 40, 8))
    rows = bn * 49
    grid = (n // bn,)
    sc = scale.reshape(1, -1)
    sh = shift.reshape(1, -1)
    body = functools.partial(_pool_body, bn=bn, rows=rows)
    return pl.pallas_call(
        body,
        grid=grid,
        in_specs=[pl.BlockSpec((rows, c), lambda i: (i, 0)),
                  pl.BlockSpec(sc.shape, lambda i: (0, 0)),
                  pl.BlockSpec(sh.shape, lambda i: (0, 0))],
        out_specs=pl.BlockSpec((bn, c), lambda i: (i, 0)),
        out_shape=jax.ShapeDtypeStruct((n, c), jnp.float32),
    )(x_flat, sc, sh)


def _cnn(node_grid, p):
    n = node_grid.shape[0]
    x = node_grid.transpose(0, 2, 3, 1).reshape(n * 49, node_grid.shape[1])
    w1 = p['c1'].transpose(2, 3, 1, 0).reshape(-1, p['c1'].shape[0])
    w2 = p['c2'].transpose(2, 3, 1, 0).reshape(-1, p['c2'].shape[0])
    w3 = p['c3'].transpose(2, 3, 1, 0).reshape(-1, p['c3'].shape[0])
    one = jnp.ones((node_grid.shape[1],), jnp.float32)
    zero = jnp.zeros((node_grid.shape[1],), jnp.float32)
    cnt = jnp.float32(n * 49)
    r1, s1 = _conv_layer(x, w1, one, zero, p['c1'].shape[0], activate=False)
    sc1, sh1 = _bn_affine(s1, p['bn1_g'], p['bn1_b'], cnt)
    r2, s2 = _conv_layer(r1, w2, sc1, sh1, p['c2'].shape[0], activate=True)
    sc2, sh2 = _bn_affine(s2, p['bn2_g'], p['bn2_b'], cnt)
    r3, s3 = _conv_layer(r2, w3, sc2, sh2, p['c3'].shape[0], activate=True)
    sc3, sh3 = _bn_affine(s3, p['bn3_g'], p['bn3_b'], cnt)
    return _pool(r3, sc3, sh3)


def _edge_mlp(edge_x, p):
    e, din = edge_x.shape
    d1 = p['eae_w1'].shape[1]
    be = _pick_block(e, (4000, 2000, 1000, 8))
    grid = (e // be,)
    full = lambda a: pl.BlockSpec(a.shape, lambda i: (0,) * a.ndim)
    vecs = [p['eae_b1'].reshape(1, -1), p['eae_g1'].reshape(1, -1),
            p['eae_e1'].reshape(1, -1), p['eae_w2'],
            p['eae_b2'].reshape(1, -1), p['eae_g2'].reshape(1, -1),
            p['eae_e2'].reshape(1, -1)]
    return pl.pallas_call(
        _node_mlp_body,
        grid=grid,
        in_specs=[pl.BlockSpec((be, din), lambda i: (i, 0)), full(p['eae_w1'])]
                 + [full(v) for v in vecs],
        out_specs=pl.BlockSpec((be, d1), lambda i: (i, 0)),
        out_shape=jax.ShapeDtypeStruct((e, d1), jnp.float32),
    )(edge_x, p['eae_w1'], *vecs)


def _concat_xs_body(h_ref, g_ref, ws_ref, x_ref, xs_ref):
    x = jnp.concatenate([h_ref[...], g_ref[...]], axis=1)
    x_ref[...] = x
    xs_ref[...] = jnp.dot(x, ws_ref[...], preferred_element_type=jnp.float32, precision=lax.Precision.HIGHEST)


def _concat_xs(h, g, ws):
    n = h.shape[0]
    d = h.shape[1] + g.shape[1]
    bn = _pick_block(n, (1000, 500, 200, 100, 8))
    grid = (n // bn,)
    return pl.pallas_call(
        _concat_xs_body,
        grid=grid,
        in_specs=[pl.BlockSpec((bn, h.shape[1]), lambda i: (i, 0)),
                  pl.BlockSpec((bn, g.shape[1]), lambda i: (i, 0)),
                  pl.BlockSpec(ws.shape, lambda i: (0, 0))],
        out_specs=[pl.BlockSpec((bn, d), lambda i: (i, 0)),
                   pl.BlockSpec((bn, d), lambda i: (i, 0))],
        out_shape=[jax.ShapeDtypeStruct((n, d), jnp.float32),
                   jax.ShapeDtypeStruct((n, d), jnp.float32)],
    )(h, g, ws)


def _msg_body(gx_ref, e_ref, we_ref, gm_ref, bm_ref, o_ref):
    t = gx_ref[...] + jnp.dot(e_ref[...], we_ref[...],
                              preferred_element_type=jnp.float32, precision=lax.Precision.HIGHEST)
    o_ref[...] = _mish(_ln(t, gm_ref[...], bm_ref[...]))


def _msg(gx, e_enc, we, gm, bm):
    e, d = gx.shape
    be = _pick_block(e, (4000, 2000, 1000, 8))
    grid = (e // be,)
    gmr = gm.reshape(1, -1)
    bmr = bm.reshape(1, -1)
    return pl.pallas_call(
        _msg_body,
        grid=grid,
        in_specs=[pl.BlockSpec((be, d), lambda i: (i, 0)),
                  pl.BlockSpec((be, e_enc.shape[1]), lambda i: (i, 0)),
                  pl.BlockSpec(we.shape, lambda i: (0, 0)),
                  pl.BlockSpec(gmr.shape, lambda i: (0, 0)),
                  pl.BlockSpec(bmr.shape, lambda i: (0, 0))],
        out_specs=pl.BlockSpec((be, d), lambda i: (i, 0)),
        out_shape=jax.ShapeDtypeStruct((e, d), jnp.float32),
    )(gx, e_enc, we, gmr, bmr)


def _update_body(x_ref, agg_ref, wu_ref, gu_ref, bu_ref, ws_ref,
                 xn_ref, xs_ref, gs_ref, *, want_xs, want_gsum):
    i = pl.program_id(0)
    u = jnp.dot(agg_ref[...], wu_ref[...], preferred_element_type=jnp.float32, precision=lax.Precision.HIGHEST)
    xn = _ln(x_ref[...] + u, gu_ref[...], bu_ref[...])
    xn_ref[...] = xn
    if want_xs:
        xs_ref[...] = jnp.dot(xn, ws_ref[...],
                              preferred_element_type=jnp.float32, precision=lax.Precision.HIGHEST)
    if want_gsum:
        d = xn.shape[1]
        cs = jnp.sum(xn, axis=0)
        row = lax.broadcasted_iota(jnp.int32, (8, d), 0)
        blockstat = jnp.where(row == 0,
                              jnp.broadcast_to(cs[None, :], (8, d)), 0.0)

        @pl.when(i == 0)
        def _():
            gs_ref[...] = jnp.zeros_like(gs_ref)

        gs_ref[...] += blockstat


def _update(x, agg, wu, gu, bu, ws_next):
    n, d = x.shape
    bn = _pick_block(n, (1000, 500, 200, 100, 8))
    grid = (n // bn,)
    want_xs = ws_next is not None
    want_gsum = not want_xs
    ws = ws_next if want_xs else wu
    gur = gu.reshape(1, -1)
    bur = bu.reshape(1, -1)
    body = functools.partial(_update_body, want_xs=want_xs,
                             want_gsum=want_gsum)
    xn, xs, gs = pl.pallas_call(
        body,
        grid=grid,
        in_specs=[pl.BlockSpec((bn, d), lambda i: (i, 0)),
                  pl.BlockSpec((bn, d), lambda i: (i, 0)),
                  pl.BlockSpec(wu.shape, lambda i: (0, 0)),
                  pl.BlockSpec(gur.shape, lambda i: (0, 0)),
                  pl.BlockSpec(bur.shape, lambda i: (0, 0)),
                  pl.BlockSpec(ws.shape, lambda i: (0, 0))],
        out_specs=[pl.BlockSpec((bn, d), lambda i: (i, 0)),
                   pl.BlockSpec((bn, d), lambda i: (i, 0)),
                   pl.BlockSpec((8, d), lambda i: (0, 0))],
        out_shape=[jax.ShapeDtypeStruct((n, d), jnp.float32),
                   jax.ShapeDtypeStruct((n, d), jnp.float32),
                   jax.ShapeDtypeStruct((8, d), jnp.float32)],
    )(x, agg, wu, gur, bur, ws)
    return xn, (xs if want_xs else None), (gs[0] if want_gsum else None)


def _heads_body(x_ref, gm_ref,
                sw1_ref, sb1_ref, sg1_ref, se1_ref, sw2_ref, sb2_ref,
                qw1_ref, qb1_ref, qg1_ref, qe1_ref, qw2_ref, qb2_ref,
                qg2_ref, qe2_ref,
                kw1_ref, kb1_ref, kg1_ref, ke1_ref, kw2_ref, kb2_ref,
                kg2_ref, ke2_ref,
                bw1_ref, bb1_ref, bg1_ref, be1_ref, bw2_ref, bb2_ref,
                bg2_ref, be2_ref, bw3_ref, bb3_ref,
                seg_ref, q_ref, k_ref, bot_ref):
    x = x_ref[...]
    bn = x.shape[0]
    lg = jnp.concatenate(
        [x, jnp.broadcast_to(gm_ref[...], (bn, gm_ref.shape[1]))], axis=1)
    s = _mish(_ln(jnp.dot(lg, sw1_ref[...], preferred_element_type=jnp.float32, precision=lax.Precision.HIGHEST)
                  + sb1_ref[...], sg1_ref[...], se1_ref[...]))
    seg_ref[...] = jnp.dot(s, sw2_ref[...],
                           preferred_element_type=jnp.float32, precision=lax.Precision.HIGHEST) + sb2_ref[...]
    q = _mish(_ln(jnp.dot(lg, qw1_ref[...], preferred_element_type=jnp.float32, precision=lax.Precision.HIGHEST)
                  + qb1_ref[...], qg1_ref[...], qe1_ref[...]))
    q_ref[...] = _ln(jnp.dot(q, qw2_ref[...],
                             preferred_element_type=jnp.float32, precision=lax.Precision.HIGHEST)
                     + qb2_ref[...], qg2_ref[...], qe2_ref[...])
    k = _mish(_ln(jnp.dot(lg, kw1_ref[...], preferred_element_type=jnp.float32, precision=lax.Precision.HIGHEST)
                  + kb1_ref[...], kg1_ref[...], ke1_ref[...]))
    k_ref[...] = _ln(jnp.dot(k, kw2_ref[...],
                             preferred_element_type=jnp.float32, precision=lax.Precision.HIGHEST)
                     + kb2_ref[...], kg2_ref[...], ke2_ref[...])
    bh = jax.nn.gelu(_ln(jnp.dot(lg, bw1_ref[...],
                                 preferred_element_type=jnp.float32, precision=lax.Precision.HIGHEST)
                         + bb1_ref[...], bg1_ref[...], be1_ref[...]))
    bh = jax.nn.gelu(_ln(jnp.dot(bh, bw2_ref[...],
                                 preferred_element_type=jnp.float32, precision=lax.Precision.HIGHEST)
                         + bb2_ref[...], bg2_ref[...], be2_ref[...]))
    bot_ref[...] = jnp.dot(bh, bw3_ref[...],
                           preferred_element_type=jnp.float32, precision=lax.Precision.HIGHEST) + bb3_ref[...]


def _heads(x, gmean, p):
    n, d = x.shape
    bn = _pick_block(n, (1000, 500, 200, 100, 8))
    grid = (n // bn,)
    gm = gmean.reshape(1, -1)

    def v(name):
        return p[name].reshape(1, -1)

    ins = [x, gm,
           p['seg_w1'], v('seg_b1'), v('seg_g1'), v('seg_e1'),
           p['seg_w2'], v('seg_b2'),
           p['wq_w1'], v('wq_b1'), v('wq_g1'), v('wq_e1'),
           p['wq_w2'], v('wq_b2'), v('wq_g2'), v('wq_e2'),
           p['wk_w1'], v('wk_b1'), v('wk_g1'), v('wk_e1'),
           p['wk_w2'], v('wk_b2'), v('wk_g2'), v('wk_e2'),
           p['bh_w1'], v('bh_b1'), v('bh_g1'), v('bh_e1'),
           p['bh_w2'], v('bh_b2'), v('bh_g2'), v('bh_e2'),
           p['bh_w3'], v('bh_b3')]
    in_specs = [pl.BlockSpec((bn, d), lambda i: (i, 0))] + [
        pl.BlockSpec(a.shape, lambda i: (0,) * a.ndim) for a in ins[1:]]
    d25 = p['seg_w2'].shape[1]
    dq = p['wq_w2'].shape[1]
    return pl.pallas_call(
        _heads_body,
        grid=grid,
        in_specs=in_specs,
        out_specs=[pl.BlockSpec((bn, d25), lambda i: (i, 0)),
                   pl.BlockSpec((bn, dq), lambda i: (i, 0)),
                   pl.BlockSpec((bn, dq), lambda i: (i, 0)),
                   pl.BlockSpec((bn, 1), lambda i: (i, 0))],
        out_shape=[jax.ShapeDtypeStruct((n, d25), jnp.float32),
                   jax.ShapeDtypeStruct((n, dq), jnp.float32),
                   jax.ShapeDtypeStruct((n, dq), jnp.float32),
                   jax.ShapeDtypeStruct((n, 1), jnp.float32)],
    )(*ins)


def _inst_body(q_ref, k_ref, o_ref):
    o_ref[...] = jax.nn.sigmoid(
        lax.dot_general(q_ref[...], k_ref[...],
                        (((1,), (1,)), ((), ())),
                        preferred_element_type=jnp.float32,
                        precision=lax.Precision.HIGHEST))


def _inst(q, k):
    n, d = q.shape
    bn = 512
    grid = (pl.cdiv(n, bn), pl.cdiv(n, bn))
    return pl.pallas_call(
        _inst_body,
        grid=grid,
        in_specs=[pl.BlockSpec((bn, d), lambda i, j: (i, 0)),
                  pl.BlockSpec((bn, d), lambda i, j: (j, 0))],
        out_specs=pl.BlockSpec((bn, bn), lambda i, j: (i, j)),
        out_shape=jax.ShapeDtypeStruct((n, n), jnp.float32),
    )(q, k)


def _gather_rows(xs, src):
    return jnp.take(xs, src, axis=0)


def _segment_sum(m, dst, n):
    return jax.ops.segment_sum(m, dst, num_segments=n)


def kernel(node_x, node_grid, edge_x, edge_index, params):
    p = params
    n = node_x.shape[0]
    h = _node_mlp(node_x, p)
    g = _cnn(node_grid, p)
    e_enc = _edge_mlp(edge_x, p)
    src = edge_index[0]
    dst = edge_index[1]
    x, xs = _concat_xs(h, g, p['l0_ws'])
    gsum = None
    for i in range(3):
        gx = _gather_rows(xs, src)
        m = _msg(gx, e_enc, p['l%d_we' % i], p['l%d_gm' % i], p['l%d_bm' % i])
        agg = _segment_sum(m, dst, n)
        ws_next = p['l%d_ws' % (i + 1)] if i < 2 else None
        x, xs, gsum = _update(x, agg, p['l%d_wu' % i], p['l%d_gu' % i],
                              p['l%d_bu' % i], ws_next)
    gmean = gsum / jnp.float32(n)
    seg_out, q, k, bottom = _heads(x, gmean, p)
    inst_out = _inst(q, k)
    return seg_out, inst_out, bottom
